# baseline (speedup 1.0000x reference)
import functools
import math

import jax
import jax.numpy as jnp
from jax.experimental import pallas as pl
from jax.experimental.pallas import tpu as pltpu

_BF = jnp.bfloat16
_NEG = -1e20


def _layernorm(x, g, b, eps):
    mean = jnp.mean(x, axis=-1, keepdims=True)
    xc = x - mean
    var = jnp.mean(xc * xc, axis=-1, keepdims=True)
    inv = jax.lax.rsqrt(var + eps)
    return xc * inv * g + b


def _softmax_ctx(s, v_b):
    m = jnp.max(s, axis=-1, keepdims=True)
    p = jnp.exp(s - m)
    l = jnp.sum(p, axis=-1, keepdims=True)
    ctx = jnp.dot(p.astype(_BF), v_b, preferred_element_type=jnp.float32)
    return ctx * pl.reciprocal(l, approx=True)


def _attention(x_f32, kv_b, keep, wq, wk, wv, wo, bo, g, b, *, num_heads, eps):
    dh = x_f32.shape[-1] // num_heads
    xb = x_f32.astype(_BF)
    ctxs = []
    for h in range(num_heads):
        sl = slice(h * dh, (h + 1) * dh)
        q = jnp.dot(xb[:, sl], wq, preferred_element_type=jnp.float32)
        k = jnp.dot(kv_b[:, sl], wk, preferred_element_type=jnp.float32)
        v = jnp.dot(kv_b[:, sl], wv, preferred_element_type=jnp.float32)
        s = jax.lax.dot_general(q.astype(_BF), k.astype(_BF),
                                (((1,), (1,)), ((), ())),
                                preferred_element_type=jnp.float32)
        s = jnp.where(keep, s, _NEG)
        ctxs.append(_softmax_ctx(s, v.astype(_BF)).astype(_BF))
    ctx_all = jnp.concatenate(ctxs, axis=1)
    out = jnp.dot(ctx_all, wo, preferred_element_type=jnp.float32)
    out = out + bo + x_f32
    return _layernorm(out, g, b, eps)


def _causal_attention(x_f32, wq, wk, wv, wo, bo, g, b, *, num_heads, eps,
                      q_tile):
    L, E = x_f32.shape
    dh = E // num_heads
    nqt = L // q_tile
    xb = x_f32.astype(_BF)

    keeps = []
    for qt in range(nqt):
        lk = (qt + 1) * q_tile
        rows = jax.lax.broadcasted_iota(jnp.int32, (q_tile, lk), 0)
        cols = jax.lax.broadcasted_iota(jnp.int32, (q_tile, lk), 1)
        keeps.append(cols <= rows + qt * q_tile)

    ctxs = []
    for h in range(num_heads):
        sl = slice(h * dh, (h + 1) * dh)
        q = jnp.dot(xb[:, sl], wq, preferred_element_type=jnp.float32)
        k = jnp.dot(xb[:, sl], wk, preferred_element_type=jnp.float32)
        v = jnp.dot(xb[:, sl], wv, preferred_element_type=jnp.float32)
        kb = k.astype(_BF)
        vb = v.astype(_BF)
        parts = []
        for qt in range(nqt):
            lk = (qt + 1) * q_tile
            qrows = q[qt * q_tile:(qt + 1) * q_tile].astype(_BF)
            s = jax.lax.dot_general(qrows, kb[:lk],
                                    (((1,), (1,)), ((), ())),
                                    preferred_element_type=jnp.float32)
            s = jnp.where(keeps[qt], s, _NEG)
            parts.append(_softmax_ctx(s, vb[:lk]))
        ctxs.append(jnp.concatenate(parts, axis=0).astype(_BF))
    ctx_all = jnp.concatenate(ctxs, axis=1)
    out = jnp.dot(ctx_all, wo, preferred_element_type=jnp.float32)
    out = out + bo + x_f32
    return _layernorm(out, g, b, eps)


def _block_kernel(tgt_ref, src_ref, smask_ref,
                  sa_wq_ref, sa_wk_ref, sa_wv_ref, sa_wo_ref, sa_bo_ref,
                  ln_g_ref, ln_b_ref,
                  ca_wq_ref, ca_wk_ref, ca_wv_ref, ca_wo_ref, ca_bo_ref,
                  n1_g_ref, n1_b_ref,
                  w1_ref, b1_ref, w2_ref, b2_ref, n2_g_ref, n2_b_ref,
                  o_ref, *, num_heads, eps):
    x0 = tgt_ref[0]
    srcb = src_ref[0]
    lt = x0.shape[0]

    q_tile = 256 if lt % 256 == 0 else lt
    x1 = _causal_attention(x0,
                           sa_wq_ref[...], sa_wk_ref[...], sa_wv_ref[...],
                           sa_wo_ref[...], sa_bo_ref[...],
                           ln_g_ref[...], ln_b_ref[...],
                           num_heads=num_heads, eps=eps, q_tile=q_tile)

    skeep = smask_ref[0] != 0.0
    x2 = _attention(x1, srcb, skeep,
                    ca_wq_ref[...], ca_wk_ref[...], ca_wv_ref[...],
                    ca_wo_ref[...], ca_bo_ref[...],
                    n1_g_ref[...], n1_b_ref[...],
                    num_heads=num_heads, eps=eps)

    h = jnp.dot(x2.astype(_BF), w1_ref[...],
                preferred_element_type=jnp.float32) + b1_ref[...]
    h = jnp.maximum(h, 0.0)
    y = jnp.dot(h.astype(_BF), w2_ref[...],
                preferred_element_type=jnp.float32) + b2_ref[...]
    z = y + x2
    o_ref[0] = _layernorm(z, n2_g_ref[...], n2_b_ref[...], eps
                          ).astype(o_ref.dtype)


def kernel(tgt, src, src_mask, tgt_mask,
           sa_wq, sa_wk, sa_wv, sa_wo, sa_bo, ln_g, ln_b,
           ca_wq, ca_wk, ca_wv, ca_wo, ca_bo, n1_g, n1_b, n2_g, n2_b,
           w1, b1, w2, b2):
    B, Lt, E = tgt.shape
    Ls = src.shape[1]
    dh = sa_wq.shape[0]
    num_heads = E // dh
    hid = w1.shape[0]
    eps = 1e-5
    scale = 1.0 / math.sqrt(E)

    sa_wq_t = (sa_wq.T * scale).astype(_BF)
    ca_wq_t = (ca_wq.T * scale).astype(_BF)
    smask = src_mask[:, 0].astype(jnp.float32)
    srcb = src.astype(_BF)

    full = lambda b: (0, 0)
    kernel_fn = functools.partial(_block_kernel, num_heads=num_heads, eps=eps)
    out = pl.pallas_call(
        kernel_fn,
        out_shape=jax.ShapeDtypeStruct((B, Lt, E), tgt.dtype),
        grid=(B,),
        in_specs=[
            pl.BlockSpec((1, Lt, E), lambda b: (b, 0, 0)),
            pl.BlockSpec((1, Ls, E), lambda b: (b, 0, 0)),
            pl.BlockSpec((1, 1, Ls), lambda b: (b, 0, 0)),
            pl.BlockSpec((dh, dh), full),
            pl.BlockSpec((dh, dh), full),
            pl.BlockSpec((dh, dh), full),
            pl.BlockSpec((E, E), full),
            pl.BlockSpec((1, E), full),
            pl.BlockSpec((1, E), full),
            pl.BlockSpec((1, E), full),
            pl.BlockSpec((dh, dh), full),
            pl.BlockSpec((dh, dh), full),
            pl.BlockSpec((dh, dh), full),
            pl.BlockSpec((E, E), full),
            pl.BlockSpec((1, E), full),
            pl.BlockSpec((1, E), full),
            pl.BlockSpec((1, E), full),
            pl.BlockSpec((E, hid), full),
            pl.BlockSpec((1, hid), full),
            pl.BlockSpec((hid, E), full),
            pl.BlockSpec((1, E), full),
            pl.BlockSpec((1, E), full),
            pl.BlockSpec((1, E), full),
        ],
        out_specs=pl.BlockSpec((1, Lt, E), lambda b: (b, 0, 0)),
        compiler_params=pltpu.CompilerParams(
            dimension_semantics=("parallel",)),
    )(tgt, srcb, smask,
      sa_wq_t, sa_wk.T.astype(_BF), sa_wv.T.astype(_BF), sa_wo.T.astype(_BF),
      sa_bo.reshape(1, E), ln_g.reshape(1, E), ln_b.reshape(1, E),
      ca_wq_t, ca_wk.T.astype(_BF), ca_wv.T.astype(_BF), ca_wo.T.astype(_BF),
      ca_bo.reshape(1, E), n1_g.reshape(1, E), n1_b.reshape(1, E),
      w1.T.astype(_BF), b1.reshape(1, hid),
      w2.T.astype(_BF), b2.reshape(1, E),
      n2_g.reshape(1, E), n2_b.reshape(1, E))
    return out


# speedup vs baseline: 1.2860x; 1.2860x over previous
import functools
import math

import jax
import jax.numpy as jnp
from jax.experimental import pallas as pl
from jax.experimental.pallas import tpu as pltpu

_BF = jnp.bfloat16


def _layernorm(x, g, b, eps):
    mean = jnp.mean(x, axis=-1, keepdims=True)
    xc = x - mean
    var = jnp.mean(xc * xc, axis=-1, keepdims=True)
    inv = jax.lax.rsqrt(var + eps)
    return xc * inv * g + b


def _softmax_ctx(s, v_b):
    dh = v_b.shape[-1]
    m = jnp.max(s, axis=-1, keepdims=True)
    p = jnp.exp(s - m)
    ve = jnp.concatenate(
        [v_b, jnp.ones((v_b.shape[0], 1), v_b.dtype)], axis=1)
    ctx_l = jnp.dot(p.astype(v_b.dtype), ve,
                    preferred_element_type=jnp.float32)
    return ctx_l[:, :dh] * pl.reciprocal(ctx_l[:, dh:dh + 1], approx=True)


def _attention(x_f32, kv_b, amask, wq, wk, wv, wo, bo, g, b, *,
               num_heads, eps):
    dh = x_f32.shape[-1] // num_heads
    xb = x_f32.astype(_BF)
    ctxs = []
    for h in range(num_heads):
        sl = slice(h * dh, (h + 1) * dh)
        q = jnp.dot(xb[:, sl], wq, preferred_element_type=jnp.float32)
        k = jnp.dot(kv_b[:, sl], wk, preferred_element_type=jnp.float32)
        v = jnp.dot(kv_b[:, sl], wv, preferred_element_type=jnp.float32)
        s = jax.lax.dot_general(q.astype(_BF), k.astype(_BF),
                                (((1,), (1,)), ((), ())),
                                preferred_element_type=jnp.float32)
        s = s + amask
        ctxs.append(_softmax_ctx(s, v.astype(_BF)).astype(_BF))
    ctx_all = jnp.concatenate(ctxs, axis=1)
    out = jnp.dot(ctx_all, wo, preferred_element_type=jnp.float32)
    out = out + bo + x_f32
    return _layernorm(out, g, b, eps)


def _block_kernel(tgt_ref, src_ref, cmask_ref, smask_ref,
                  sa_wq_ref, sa_wk_ref, sa_wv_ref, sa_wo_ref, sa_bo_ref,
                  ln_g_ref, ln_b_ref,
                  ca_wq_ref, ca_wk_ref, ca_wv_ref, ca_wo_ref, ca_bo_ref,
                  n1_g_ref, n1_b_ref,
                  w1_ref, b1_ref, w2_ref, b2_ref, n2_g_ref, n2_b_ref,
                  o_ref, *, num_heads, eps):
    x0 = tgt_ref[0]
    srcb = src_ref[0]

    x1 = _attention(x0, x0.astype(_BF), cmask_ref[...],
                    sa_wq_ref[...], sa_wk_ref[...], sa_wv_ref[...],
                    sa_wo_ref[...], sa_bo_ref[...],
                    ln_g_ref[...], ln_b_ref[...],
                    num_heads=num_heads, eps=eps)

    x2 = _attention(x1, srcb, smask_ref[0],
                    ca_wq_ref[...], ca_wk_ref[...], ca_wv_ref[...],
                    ca_wo_ref[...], ca_bo_ref[...],
                    n1_g_ref[...], n1_b_ref[...],
                    num_heads=num_heads, eps=eps)

    h = jnp.dot(x2.astype(_BF), w1_ref[...],
                preferred_element_type=jnp.float32) + b1_ref[...]
    h = jnp.maximum(h, 0.0)
    y = jnp.dot(h.astype(_BF), w2_ref[...],
                preferred_element_type=jnp.float32) + b2_ref[...]
    z = y + x2
    o_ref[0] = _layernorm(z, n2_g_ref[...], n2_b_ref[...], eps
                          ).astype(o_ref.dtype)


def kernel(tgt, src, src_mask, tgt_mask,
           sa_wq, sa_wk, sa_wv, sa_wo, sa_bo, ln_g, ln_b,
           ca_wq, ca_wk, ca_wv, ca_wo, ca_bo, n1_g, n1_b, n2_g, n2_b,
           w1, b1, w2, b2):
    B, Lt, E = tgt.shape
    Ls = src.shape[1]
    dh = sa_wq.shape[0]
    num_heads = E // dh
    hid = w1.shape[0]
    eps = 1e-5
    scale = 1.0 / math.sqrt(E)

    sa_wq_t = (sa_wq.T * scale).astype(_BF)
    ca_wq_t = (ca_wq.T * scale).astype(_BF)
    cmask = (tgt_mask[0, 0].astype(jnp.float32) - 1.0) * 1e20
    smask = (src_mask[:, 0].astype(jnp.float32) - 1.0) * 1e20
    srcb = src.astype(_BF)

    full = lambda b: (0, 0)
    kernel_fn = functools.partial(_block_kernel, num_heads=num_heads, eps=eps)
    out = pl.pallas_call(
        kernel_fn,
        out_shape=jax.ShapeDtypeStruct((B, Lt, E), tgt.dtype),
        grid=(B,),
        in_specs=[
            pl.BlockSpec((1, Lt, E), lambda b: (b, 0, 0)),
            pl.BlockSpec((1, Ls, E), lambda b: (b, 0, 0)),
            pl.BlockSpec((Lt, Lt), full),
            pl.BlockSpec((1, 1, Ls), lambda b: (b, 0, 0)),
            pl.BlockSpec((dh, dh), full),
            pl.BlockSpec((dh, dh), full),
            pl.BlockSpec((dh, dh), full),
            pl.BlockSpec((E, E), full),
            pl.BlockSpec((1, E), full),
            pl.BlockSpec((1, E), full),
            pl.BlockSpec((1, E), full),
            pl.BlockSpec((dh, dh), full),
            pl.BlockSpec((dh, dh), full),
            pl.BlockSpec((dh, dh), full),
            pl.BlockSpec((E, E), full),
            pl.BlockSpec((1, E), full),
            pl.BlockSpec((1, E), full),
            pl.BlockSpec((1, E), full),
            pl.BlockSpec((E, hid), full),
            pl.BlockSpec((1, hid), full),
            pl.BlockSpec((hid, E), full),
            pl.BlockSpec((1, E), full),
            pl.BlockSpec((1, E), full),
            pl.BlockSpec((1, E), full),
        ],
        out_specs=pl.BlockSpec((1, Lt, E), lambda b: (b, 0, 0)),
        compiler_params=pltpu.CompilerParams(
            dimension_semantics=("parallel",)),
    )(tgt, srcb, cmask, smask,
      sa_wq_t, sa_wk.T.astype(_BF), sa_wv.T.astype(_BF), sa_wo.T.astype(_BF),
      sa_bo.reshape(1, E), ln_g.reshape(1, E), ln_b.reshape(1, E),
      ca_wq_t, ca_wk.T.astype(_BF), ca_wv.T.astype(_BF), ca_wo.T.astype(_BF),
      ca_bo.reshape(1, E), n1_g.reshape(1, E), n1_b.reshape(1, E),
      w1.T.astype(_BF), b1.reshape(1, hid),
      w2.T.astype(_BF), b2.reshape(1, E),
      n2_g.reshape(1, E), n2_b.reshape(1, E))
    return out
